# baseline (speedup 1.0000x reference)
import functools

import jax
import jax.numpy as jnp
from jax import lax
from jax.experimental import pallas as pl
from jax.experimental.pallas import tpu as pltpu
from jax.experimental.pallas import tpu_sc as plsc

N = 10000
NP = 10240
D = 256
DQ = 64
NQ = 4
NC = 2
NS = 16
CHUNK = 128
NB = 4
HC = 80
E_PAD_MULT = 81920
RPT = NP // NS

_sc_mesh = plsc.VectorSubcoreMesh(core_axis_name="c", subcore_axis_name="s")


def _deg_body(ep, dst_hbm, degp_hbm, deg_acc, zbuf, ones_buf,
              d0, d1, d2, d3, sd0, sd1, sd2, sd3):
    c = lax.axis_index("c")
    s = lax.axis_index("s")

    def fill(i, _):
        zbuf[i, :] = jnp.zeros((16,), jnp.float32)
        ones_buf[i, :] = jnp.ones((16,), jnp.float32)
        return 0

    lax.fori_loop(0, CHUNK, fill, 0)

    def fillz(i, _):
        zbuf[i, :] = jnp.zeros((16,), jnp.float32)
        return 0

    lax.fori_loop(CHUNK, RPT, fillz, 0)

    pltpu.sync_copy(zbuf, deg_acc.at[pl.ds(s * RPT, RPT)])
    plsc.subcore_barrier()

    cpt = ep // (NC * NS * CHUNK)
    base = (s * NC + c) * cpt * CHUNK
    dbufs = (d0, d1, d2, d3)
    dsems = (sd0, sd1, sd2, sd3)
    for b in range(NB):
        pltpu.async_copy(dst_hbm.at[pl.ds(base + b * CHUNK, CHUNK)],
                         dbufs[b], dsems[b])

    def body(j, _):
        for b in range(NB):
            k = j * NB + b
            pltpu.make_async_copy(dst_hbm.at[pl.ds(0, CHUNK)], dbufs[b],
                                  dsems[b]).wait()
            pltpu.sync_copy(ones_buf, deg_acc.at[dbufs[b]], add=True)
            knext = k + NB

            @pl.when(knext < cpt)
            def _():
                off = pl.multiple_of(base + knext * CHUNK, CHUNK)
                pltpu.async_copy(dst_hbm.at[pl.ds(off, CHUNK)], dbufs[b],
                                 dsems[b])

        return 0

    lax.fori_loop(0, cpt // NB, body, 0)
    plsc.subcore_barrier()
    pltpu.sync_copy(deg_acc.at[pl.ds(s * RPT, RPT)],
                    degp_hbm.at[pl.ds(c * NP + s * RPT, RPT)])


def _deg_call(dst_p, ep):
    kfn = pl.kernel(
        functools.partial(_deg_body, ep),
        out_type=jax.ShapeDtypeStruct((NC * NP, 16), jnp.float32),
        mesh=_sc_mesh,
        scratch_types=[
            pltpu.VMEM_SHARED((NP, 16), jnp.float32),
            pltpu.VMEM((RPT, 16), jnp.float32),
            pltpu.VMEM((CHUNK, 16), jnp.float32),
            pltpu.VMEM((CHUNK,), jnp.int32),
            pltpu.VMEM((CHUNK,), jnp.int32),
            pltpu.VMEM((CHUNK,), jnp.int32),
            pltpu.VMEM((CHUNK,), jnp.int32),
            pltpu.SemaphoreType.DMA,
            pltpu.SemaphoreType.DMA,
            pltpu.SemaphoreType.DMA,
            pltpu.SemaphoreType.DMA,
        ],
    )
    return kfn(dst_p)


def _prop_body(ep, src_hbm, dst_hbm, y_hbm, r_hbm, out_hbm, y_sp,
               acc, src_all, dst_all, rows, r_buf,
               sg0, sg1, sg2, sg3, ss0, ss1, ss2, ss3):
    c = lax.axis_index("c")
    s = lax.axis_index("s")

    ept = ep // NS
    cpt = ept // HC
    pltpu.sync_copy(src_hbm.at[pl.ds(s * ept, ept)], src_all)
    pltpu.sync_copy(dst_hbm.at[pl.ds(s * ept, ept)], dst_all)

    gsems = (sg0, sg1, sg2, sg3)
    ssems = (ss0, ss1, ss2, ss3)

    def issue_g(k, b):
        soff = pl.multiple_of(k * HC, HC)
        pltpu.async_copy(y_sp.at[src_all.at[pl.ds(soff, HC)]],
                         rows.at[b], gsems[b])

    def run_ring():
        for b in range(2):
            issue_g(b, b)

        def body(j, _):
            for b in range(NB):
                k = j * NB + b
                b2 = (b + 2) % NB
                pltpu.make_async_copy(y_hbm.at[pl.ds(0, HC)], rows.at[b],
                                      gsems[b]).wait()
                koff = pl.multiple_of(k * HC, HC)
                pltpu.async_copy(rows.at[b],
                                 acc.at[dst_all.at[pl.ds(koff, HC)]],
                                 ssems[b], add=True)

                @pl.when(k >= 2)
                def _():
                    pltpu.make_async_copy(rows.at[b2], acc.at[pl.ds(0, HC)],
                                          ssems[b2]).wait()

                @pl.when(k + 2 < cpt)
                def _():
                    issue_g(k + 2, b2)

            return 0

        lax.fori_loop(0, cpt // NB, body, 0)
        for b in (2, 3):
            pltpu.make_async_copy(rows.at[b], acc.at[pl.ds(0, HC)],
                                  ssems[b]).wait()
        plsc.subcore_barrier()

    CR = 80
    for q in range(NC):
        base_row = (NC * c + q) * NP + s * RPT
        pltpu.sync_copy(y_hbm.at[pl.ds(base_row, RPT)],
                        y_sp.at[pl.ds(s * RPT, RPT)])
        pltpu.sync_copy(y_hbm.at[pl.ds(base_row, RPT)],
                        acc.at[pl.ds(s * RPT, RPT)])
        plsc.subcore_barrier()

        run_ring()

        for t in range(RPT // CR):
            r0 = t * CR
            pltpu.sync_copy(acc.at[pl.ds(s * RPT + r0, CR)], rows.at[0])
            pltpu.sync_copy(r_hbm.at[pl.ds(s * RPT + r0, CR)], r_buf)

            def scale_row(i, _):
                rvec = r_buf[i, :]
                for f in range(DQ // 16):
                    rows[0, i, pl.ds(f * 16, 16)] = (
                        rows[0, i, pl.ds(f * 16, 16)] * rvec)
                return 0

            lax.fori_loop(0, CR, scale_row, 0)
            pltpu.sync_copy(rows.at[0], y_sp.at[pl.ds(s * RPT + r0, CR)])
            pltpu.sync_copy(rows.at[0], acc.at[pl.ds(s * RPT + r0, CR)])
        plsc.subcore_barrier()

        run_ring()

        pltpu.sync_copy(acc.at[pl.ds(s * RPT, RPT)],
                        out_hbm.at[pl.ds(base_row, RPT)])


def _prop_call(src_p, dst_p, y_flat, r_vec, ep):
    kfn = pl.kernel(
        _prop_body if False else functools.partial(_prop_body, ep),
        out_type=jax.ShapeDtypeStruct((NQ * NP, DQ), jnp.float32),
        mesh=_sc_mesh,
        scratch_types=[
            pltpu.VMEM_SHARED((NP, DQ), jnp.float32),
            pltpu.VMEM_SHARED((NP, DQ), jnp.float32),
            pltpu.VMEM((ep // NS,), jnp.int32),
            pltpu.VMEM((ep // NS,), jnp.int32),
            pltpu.VMEM((NB, HC, DQ), jnp.float32),
            pltpu.VMEM((80, 16), jnp.float32),
        ] + [pltpu.SemaphoreType.DMA] * 8,
        compiler_params=pltpu.CompilerParams(use_tc_tiling_on_sc=False),
    )
    return kfn(src_p, dst_p, y_flat, r_vec)


_RB = 2560


def _scale_body(x_ref, degp_ref, y_ref):
    dinv = lax.rsqrt(degp_ref[0, :, 0:1] + degp_ref[1, :, 0:1] + 1.0)
    for q in range(NQ):
        y_ref[q, :, :] = x_ref[:, q * DQ:(q + 1) * DQ] * dinv


def _scale_call(xp, degp):
    return pl.pallas_call(
        _scale_body,
        grid=(NP // _RB,),
        in_specs=[
            pl.BlockSpec((_RB, D), lambda i: (i, 0)),
            pl.BlockSpec((NC, _RB, 16), lambda i: (0, i, 0)),
        ],
        out_specs=pl.BlockSpec((NQ, _RB, DQ), lambda i: (0, i, 0)),
        out_shape=jax.ShapeDtypeStruct((NQ, NP, DQ), jnp.float32),
    )(xp, degp)


def _mid_body(acc_ref, degp_ref, y_ref):
    r = 1.0 / (degp_ref[0, :, 0:1] + degp_ref[1, :, 0:1] + 1.0)
    for q in range(NQ):
        y_ref[q, :, :] = acc_ref[q, :, :] * r


def _mid_call(acc, degp):
    return pl.pallas_call(
        _mid_body,
        grid=(NP // _RB,),
        in_specs=[
            pl.BlockSpec((NQ, _RB, DQ), lambda i: (0, i, 0)),
            pl.BlockSpec((NC, _RB, 16), lambda i: (0, i, 0)),
        ],
        out_specs=pl.BlockSpec((NQ, _RB, DQ), lambda i: (0, i, 0)),
        out_shape=jax.ShapeDtypeStruct((NQ, NP, DQ), jnp.float32),
    )(acc, degp)


def _final_body(acc_ref, degp_ref, w_ref, b_ref, out_ref):
    dinv = lax.rsqrt(degp_ref[0, :, 0:1] + degp_ref[1, :, 0:1] + 1.0)
    h = jnp.concatenate([acc_ref[q, :, :] * dinv for q in range(NQ)], axis=1)
    z = jnp.dot(h, w_ref[...], preferred_element_type=jnp.float32) + b_ref[...]
    m = jnp.max(z, axis=1, keepdims=True)
    lse = jnp.log(jnp.sum(jnp.exp(z - m), axis=1, keepdims=True)) + m
    out_ref[...] = z - lse


def _final_call(acc, degp, W, b2):
    return pl.pallas_call(
        _final_body,
        grid=(NP // _RB,),
        in_specs=[
            pl.BlockSpec((NQ, _RB, DQ), lambda i: (0, i, 0)),
            pl.BlockSpec((NC, _RB, 16), lambda i: (0, i, 0)),
            pl.BlockSpec((D, D), lambda i: (0, 0)),
            pl.BlockSpec((1, D), lambda i: (0, 0)),
        ],
        out_specs=pl.BlockSpec((_RB, D), lambda i: (i, 0)),
        out_shape=jax.ShapeDtypeStruct((NP, D), jnp.float32),
    )(acc, degp, W, b2)


def kernel(x, edge_index, W, b):
    e = edge_index.shape[1]
    ep = ((e + E_PAD_MULT - 1) // E_PAD_MULT) * E_PAD_MULT
    src = edge_index[0].astype(jnp.int32)
    dst = edge_index[1].astype(jnp.int32)
    pad = ep - e
    src_p = jnp.concatenate([src, jnp.zeros((pad,), jnp.int32)])
    dst_p = jnp.concatenate([dst, jnp.full((pad,), N, jnp.int32)])

    degp_flat = _deg_call(dst_p, ep)
    degp = degp_flat.reshape(NC, NP, 16)

    xp = jnp.pad(x, ((0, NP - N), (0, 0)))
    y = _scale_call(xp, degp)

    r_vec = jnp.broadcast_to(
        (1.0 / (degp[0, :, 0] + degp[1, :, 0] + 1.0))[:, None], (NP, 16))
    acc2 = _prop_call(src_p, dst_p, y.reshape(NQ * NP, DQ), r_vec, ep)

    out = _final_call(acc2.reshape(NQ, NP, DQ), degp, W, b.reshape(1, D))
    return out[:N]
